# baseline (speedup 1.0000x reference)
import functools

import jax
import jax.numpy as jnp
from jax import lax
from jax.experimental import pallas as pl
from jax.experimental.pallas import tpu as pltpu
from jax.experimental.pallas import tpu_sc as plsc

LANES = 16


def _sc_body(hidden, ntypes, num_edges, chunk, num_heads, num_cores,
             emb_hbm, bias_hbm, idx_hbm, out_hbm,
             emb_v, bias_v, idx_v, val_v, red_v, isem, osem, wsem):
    wid = lax.axis_index("s") * num_cores + lax.axis_index("c")
    chunks_per_b = num_edges // chunk
    b = wid // chunks_per_b
    off = pl.multiple_of((wid % chunks_per_b) * chunk, 8)

    idx_cp = pltpu.async_copy(
        idx_hbm.at[pl.ds(b, 1), pl.ds(off, chunk)], idx_v, isem)

    emb_cp = pltpu.async_copy(emb_hbm, emb_v, wsem)
    bias_cp = pltpu.async_copy(bias_hbm, bias_v, wsem)
    emb_cp.wait()
    bias_cp.wait()

    def dot_body(j, accs):
        start = pl.multiple_of(j * LANES, LANES)
        bv = bias_v[0, pl.ds(start, LANES)]
        return tuple(accs[t] + emb_v[t, pl.ds(start, LANES)] * bv
                     for t in range(ntypes))

    accs = lax.fori_loop(
        0, hidden // LANES, dot_body,
        tuple(jnp.zeros((LANES,), jnp.float32) for _ in range(ntypes)),
        unroll=4)

    lane_ids = jnp.arange(LANES, dtype=jnp.int32)
    svecs = []
    for t in range(ntypes):
        a = accs[t]
        for stride in (1, 2, 4, 8):
            red_v[...] = a
            a = a + plsc.load_gather(red_v, [lane_ids ^ stride])
        svecs.append(a)

    vt = svecs[ntypes - 1]
    for t in range(ntypes - 2, -1, -1):
        vt = jnp.where(lane_ids == t, svecs[t], vt)
    red_v[...] = vt

    idx_cp.wait()

    nsub = 2
    sub = chunk // nsub
    copies = []
    for si in range(nsub):
        sbase = si * sub

        def body(i, carry, sbase=sbase):
            start = pl.multiple_of(sbase + i * LANES, LANES)
            tv = idx_v[0, pl.ds(start, LANES)]
            val_v[0, 0, pl.ds(start, LANES)] = plsc.load_gather(red_v, [tv])
            return carry

        lax.fori_loop(0, sub // LANES, body, 0, unroll=4)
        for h in range(num_heads):
            copies.append(pltpu.async_copy(
                val_v.at[pl.ds(0, 1), pl.ds(0, 1), pl.ds(sbase, sub)],
                out_hbm.at[pl.ds(b, 1), pl.ds(h, 1), pl.ds(off + sbase, sub)],
                osem))
    for c in copies:
        c.wait()


def kernel(query, edge_types, emb_weight, bias_weight):
    B, H = query.shape[0], query.shape[1]
    ntypes, hidden = emb_weight.shape
    num_edges = edge_types.shape[1]

    info = plsc.get_sparse_core_info()
    nw = info.num_cores * info.num_subcores
    chunk = (B * num_edges) // nw

    idx = edge_types.astype(jnp.int32)

    mesh = plsc.VectorSubcoreMesh(core_axis_name="c", subcore_axis_name="s")
    body = functools.partial(_sc_body, hidden, ntypes, num_edges, chunk, H,
                             info.num_cores)
    return pl.kernel(
        body,
        out_type=jax.ShapeDtypeStruct((B, H, num_edges), jnp.float32),
        mesh=mesh,
        compiler_params=pltpu.CompilerParams(needs_layout_passes=False),
        scratch_types=[
            pltpu.VMEM((ntypes, hidden), jnp.float32),
            pltpu.VMEM((1, hidden), jnp.float32),
            pltpu.VMEM((1, chunk), jnp.int32),
            pltpu.VMEM((1, 1, chunk), jnp.float32),
            pltpu.VMEM((LANES,), jnp.float32),
            pltpu.SemaphoreType.DMA,
            pltpu.SemaphoreType.DMA,
            pltpu.SemaphoreType.DMA,
        ],
    )(emb_weight, bias_weight, idx)


# speedup vs baseline: 1.0733x; 1.0733x over previous
import functools

import jax
import jax.numpy as jnp
from jax import lax
from jax.experimental import pallas as pl
from jax.experimental.pallas import tpu as pltpu
from jax.experimental.pallas import tpu_sc as plsc

LANES = 16


def _sc_body(hidden, ntypes, num_edges, chunk, num_heads, num_cores,
             emb_hbm, bias_hbm, idx_hbm, out_hbm,
             emb_v, bias_v, idx_v, val_v, red_v, isem, osem, wsem):
    wid = lax.axis_index("s") * num_cores + lax.axis_index("c")
    chunks_per_b = num_edges // chunk
    b = wid // chunks_per_b
    off = pl.multiple_of((wid % chunks_per_b) * chunk, 8)

    idx_cp = pltpu.async_copy(
        idx_hbm.at[pl.ds(b, 1), pl.ds(off, chunk)], idx_v, isem)

    emb_cp = pltpu.async_copy(emb_hbm, emb_v, wsem)
    bias_cp = pltpu.async_copy(bias_hbm, bias_v, wsem)
    emb_cp.wait()
    bias_cp.wait()

    def dot_body(j, accs):
        start = pl.multiple_of(j * LANES, LANES)
        bv = bias_v[0, pl.ds(start, LANES)]
        return tuple(accs[t] + emb_v[t, pl.ds(start, LANES)] * bv
                     for t in range(ntypes))

    accs = lax.fori_loop(
        0, hidden // LANES, dot_body,
        tuple(jnp.zeros((LANES,), jnp.float32) for _ in range(ntypes)),
        unroll=4)

    lane_ids = jnp.arange(LANES, dtype=jnp.int32)
    svecs = []
    for t in range(ntypes):
        a = accs[t]
        for stride in (1, 2, 4, 8):
            red_v[...] = a
            a = a + plsc.load_gather(red_v, [lane_ids ^ stride])
        svecs.append(a)

    vt = svecs[ntypes - 1]
    for t in range(ntypes - 2, -1, -1):
        vt = jnp.where(lane_ids == t, svecs[t], vt)
    red_v[...] = vt

    idx_cp.wait()

    nsub = 4
    sub = chunk // nsub
    for si in range(nsub):
        sbase = si * sub

        def body(i, carry, sbase=sbase):
            start = pl.multiple_of(sbase + i * LANES, LANES)
            tv = idx_v[0, pl.ds(start, LANES)]
            val_v[0, 0, pl.ds(start, LANES)] = plsc.load_gather(red_v, [tv])
            return carry

        lax.fori_loop(0, sub // LANES, body, 0, unroll=4)

        def hcopy(h, carry, sbase=sbase):
            pltpu.async_copy(
                val_v.at[pl.ds(0, 1), pl.ds(0, 1), pl.ds(sbase, sub)],
                out_hbm.at[pl.ds(b, 1), pl.ds(h, 1), pl.ds(off + sbase, sub)],
                osem)
            return carry

        lax.fori_loop(0, num_heads, hcopy, 0)

    def drain(i, carry):
        pltpu.make_async_copy(
            val_v.at[pl.ds(0, 1), pl.ds(0, 1), pl.ds(0, sub)],
            out_hbm.at[pl.ds(b, 1), pl.ds(0, 1), pl.ds(off, sub)],
            osem).wait()
        return carry

    lax.fori_loop(0, nsub * num_heads, drain, 0)


def kernel(query, edge_types, emb_weight, bias_weight):
    B, H = query.shape[0], query.shape[1]
    ntypes, hidden = emb_weight.shape
    num_edges = edge_types.shape[1]

    info = plsc.get_sparse_core_info()
    nw = info.num_cores * info.num_subcores
    chunk = (B * num_edges) // nw

    idx = edge_types.astype(jnp.int32)

    mesh = plsc.VectorSubcoreMesh(core_axis_name="c", subcore_axis_name="s")
    body = functools.partial(_sc_body, hidden, ntypes, num_edges, chunk, H,
                             info.num_cores)
    return pl.kernel(
        body,
        out_type=jax.ShapeDtypeStruct((B, H, num_edges), jnp.float32),
        mesh=mesh,
        compiler_params=pltpu.CompilerParams(needs_layout_passes=False),
        scratch_types=[
            pltpu.VMEM((ntypes, hidden), jnp.float32),
            pltpu.VMEM((1, hidden), jnp.float32),
            pltpu.VMEM((1, chunk), jnp.int32),
            pltpu.VMEM((1, 1, chunk), jnp.float32),
            pltpu.VMEM((LANES,), jnp.float32),
            pltpu.SemaphoreType.DMA,
            pltpu.SemaphoreType.DMA,
            pltpu.SemaphoreType.DMA,
        ],
    )(emb_weight, bias_weight, idx)
